# baseline (speedup 1.0000x reference)
import functools

import jax
import jax.numpy as jnp
from jax import lax
from jax.experimental import pallas as pl
from jax.experimental.pallas import tpu as pltpu
from jax.experimental.pallas import tpu_sc as plsc

_B = 16384
_D = 16
_NC = 2
_NS = 16
_NW = _NC * _NS
_BPW = _B // _NW
_CH = 128
_NCH = _BPW // _CH
_L = 16
_NBLK = _BPW // _L
_SHARP = 5.0


def _body(tbl, rec, u, out, recv, uv, vrows, urows, outv, sem_v, sem_u):
    wid = lax.axis_index("s") * _NC + lax.axis_index("c")
    base = wid * _BPW

    for i in range(_NCH):
        pltpu.sync_copy(rec.at[pl.ds(base + i * _CH, _CH)], recv.at[i])
        pltpu.sync_copy(u.at[pl.ds(base + i * _CH, _CH)], uv.at[i])

    copies = []
    for i in range(_NCH):
        copies.append(
            pltpu.async_copy(tbl.at[recv.at[i]], vrows.at[pl.ds(i * _CH, _CH)], sem_v))
        copies.append(
            pltpu.async_copy(tbl.at[uv.at[i]], urows.at[pl.ds(i * _CH, _CH)], sem_u))
    for c in copies:
        c.wait()

    lanes = lax.iota(jnp.int32, 16)

    def blk(j, carry):
        rows_idx = j * _L + lanes
        acc = jnp.zeros((_L,), jnp.float32)
        for d in range(_D):
            col = jnp.full((_L,), d, jnp.int32)
            vcol = plsc.load_gather(vrows, [rows_idx, col])
            ucol = plsc.load_gather(urows, [rows_idx, col])
            acc = acc + vcol * ucol
        sig = 1.0 / (1.0 + jnp.exp(-_SHARP * acc))
        outv[pl.ds(j * _L, _L)] = sig
        return carry

    lax.fori_loop(0, _NBLK, blk, 0)

    pltpu.sync_copy(outv, out.at[pl.ds(base, _BPW)])


def kernel(uEmb, vEmb, rec, u):
    rec = rec.astype(jnp.int32)
    u = u.astype(jnp.int32) + jnp.int32(vEmb.shape[0])
    tbl = jnp.concatenate([vEmb, uEmb], axis=0)
    mesh = plsc.VectorSubcoreMesh(core_axis_name="c", subcore_axis_name="s")
    f = pl.kernel(
        _body,
        mesh=mesh,
        out_type=jax.ShapeDtypeStruct((_B,), jnp.float32),
        scratch_types=[
            pltpu.VMEM((_NCH, _CH), jnp.int32),
            pltpu.VMEM((_NCH, _CH), jnp.int32),
            pltpu.VMEM((_BPW, _D), jnp.float32),
            pltpu.VMEM((_BPW, _D), jnp.float32),
            pltpu.VMEM((_BPW,), jnp.float32),
            pltpu.SemaphoreType.DMA,
            pltpu.SemaphoreType.DMA,
        ],
        compiler_params=pltpu.CompilerParams(
            needs_layout_passes=False, use_tc_tiling_on_sc=False),
    )
    return f(tbl, rec, u)


# speedup vs baseline: 1.6293x; 1.6293x over previous
import functools

import jax
import jax.numpy as jnp
from jax import lax
from jax.experimental import pallas as pl
from jax.experimental.pallas import tpu as pltpu
from jax.experimental.pallas import tpu_sc as plsc

_B = 16384
_D = 16
_G = 8
_NC = 2
_NS = 16
_NW = _NC * _NS
_BPW = _B // _NW
_HP = 16
_HPW = _BPW // _HP
_L = 16
_NBLK = _HPW // _L
_SHARP = 5.0


def _body(u_emb, v_emb, rec, u, out, recv, uv, vgrp, ugrp, outv, sem_v, sem_u):
    wid = lax.axis_index("s") * _NC + lax.axis_index("c")
    base = wid * _BPW

    pltpu.sync_copy(rec.at[pl.ds(base, _BPW)], recv)
    pltpu.sync_copy(u.at[pl.ds(base, _BPW)], uv)

    lanes = lax.iota(jnp.int32, 16)

    for h in range(_HP):
        hbase = h * _HPW

        def fire(jb, carry):
            rvec = recv[pl.ds(hbase + jb * _L, _L)]
            qvec = uv[pl.ds(hbase + jb * _L, _L)]
            for k in range(_L):
                slot = jb * _L + k
                g8v = rvec[k] & jnp.int32(~(_G - 1))
                g8u = qvec[k] & jnp.int32(~(_G - 1))
                pltpu.async_copy(
                    v_emb.at[pl.ds(pl.multiple_of(g8v, _G), _G), :],
                    vgrp.at[pl.ds(slot * _G, _G), :], sem_v)
                pltpu.async_copy(
                    u_emb.at[pl.ds(pl.multiple_of(g8u, _G), _G), :],
                    ugrp.at[pl.ds(slot * _G, _G), :], sem_u)
            return carry

        lax.fori_loop(0, _NBLK, fire, 0)

        def drain(jb, carry):
            for k in range(_L):
                slot = jb * _L + k
                pltpu.make_async_copy(
                    v_emb.at[pl.ds(0, _G), :],
                    vgrp.at[pl.ds(slot * _G, _G), :], sem_v).wait()
                pltpu.make_async_copy(
                    u_emb.at[pl.ds(0, _G), :],
                    ugrp.at[pl.ds(slot * _G, _G), :], sem_u).wait()
            return carry

        lax.fori_loop(0, _NBLK, drain, 0)

        def blk(jb, carry):
            rvec = recv[pl.ds(hbase + jb * _L, _L)]
            qvec = uv[pl.ds(hbase + jb * _L, _L)]
            slotbase = (jb * _L + lanes) * _G
            vrow = slotbase + (rvec & (_G - 1))
            urow = slotbase + (qvec & (_G - 1))
            acc = jnp.zeros((_L,), jnp.float32)
            for d in range(_D):
                col = jnp.full((_L,), d, jnp.int32)
                vcol = plsc.load_gather(vgrp, [vrow, col])
                ucol = plsc.load_gather(ugrp, [urow, col])
                acc = acc + vcol * ucol
            sig = 1.0 / (1.0 + jnp.exp(-_SHARP * acc))
            outv[pl.ds(hbase + jb * _L, _L)] = sig
            return carry

        lax.fori_loop(0, _NBLK, blk, 0)

    pltpu.sync_copy(outv, out.at[pl.ds(base, _BPW)])


def kernel(uEmb, vEmb, rec, u):
    rec = rec.astype(jnp.int32)
    u = u.astype(jnp.int32)
    mesh = plsc.VectorSubcoreMesh(core_axis_name="c", subcore_axis_name="s")
    f = pl.kernel(
        _body,
        mesh=mesh,
        out_type=jax.ShapeDtypeStruct((_B,), jnp.float32),
        scratch_types=[
            pltpu.VMEM((_BPW,), jnp.int32),
            pltpu.VMEM((_BPW,), jnp.int32),
            pltpu.VMEM((_HPW * _G, _D), jnp.float32),
            pltpu.VMEM((_HPW * _G, _D), jnp.float32),
            pltpu.VMEM((_BPW,), jnp.float32),
            pltpu.SemaphoreType.DMA,
            pltpu.SemaphoreType.DMA,
        ],
        compiler_params=pltpu.CompilerParams(
            needs_layout_passes=False, use_tc_tiling_on_sc=True),
    )
    return f(uEmb, vEmb, rec, u)


# speedup vs baseline: 1.6848x; 1.0341x over previous
import functools

import jax
import jax.numpy as jnp
from jax import lax
from jax.experimental import pallas as pl
from jax.experimental.pallas import tpu as pltpu
from jax.experimental.pallas import tpu_sc as plsc

_B = 16384
_D = 16
_G = 8
_NC = 2
_NS = 16
_NW = _NC * _NS
_BPW = _B // _NW
_PW = 16
_NPASS = _BPW // _PW
_SETROWS = _PW * _G
_L = 16
_SHARP = 5.0


def _body(u_emb, v_emb, rec, u, out, recv, uv, vgrp, ugrp, outv,
          sem_v0, sem_v1, sem_u0, sem_u1):
    wid = lax.axis_index("s") * _NC + lax.axis_index("c")
    base = wid * _BPW

    pltpu.sync_copy(rec.at[pl.ds(base, _BPW)], recv)
    pltpu.sync_copy(u.at[pl.ds(base, _BPW)], uv)

    lanes = lax.iota(jnp.int32, 16)
    sems = ((sem_v0, sem_u0), (sem_v1, sem_u1))

    def fire(h, parity):
        sem_v, sem_u = sems[parity]
        setbase = parity * _SETROWS
        rvec = recv[pl.ds(pl.multiple_of(h * _PW, _PW), _PW)]
        qvec = uv[pl.ds(pl.multiple_of(h * _PW, _PW), _PW)]
        for k in range(_PW):
            g8v = rvec[k] & jnp.int32(~(_G - 1))
            g8u = qvec[k] & jnp.int32(~(_G - 1))
            pltpu.async_copy(
                v_emb.at[pl.ds(pl.multiple_of(g8v, _G), _G), :],
                vgrp.at[pl.ds(setbase + k * _G, _G), :], sem_v)
            pltpu.async_copy(
                u_emb.at[pl.ds(pl.multiple_of(g8u, _G), _G), :],
                ugrp.at[pl.ds(setbase + k * _G, _G), :], sem_u)

    def drain(parity):
        sem_v, sem_u = sems[parity]
        setbase = parity * _SETROWS
        for k in range(_PW):
            pltpu.make_async_copy(
                v_emb.at[pl.ds(0, _G), :],
                vgrp.at[pl.ds(setbase + k * _G, _G), :], sem_v).wait()
            pltpu.make_async_copy(
                u_emb.at[pl.ds(0, _G), :],
                ugrp.at[pl.ds(setbase + k * _G, _G), :], sem_u).wait()

    def compute(h, parity):
        setbase = parity * _SETROWS
        rvec = recv[pl.ds(pl.multiple_of(h * _PW, _PW), _PW)]
        qvec = uv[pl.ds(pl.multiple_of(h * _PW, _PW), _PW)]
        slotbase = setbase + lanes * _G
        vrow = slotbase + (rvec & (_G - 1))
        urow = slotbase + (qvec & (_G - 1))
        acc = jnp.zeros((_L,), jnp.float32)
        for d in range(_D):
            col = jnp.full((_L,), d, jnp.int32)
            vcol = plsc.load_gather(vgrp, [vrow, col])
            ucol = plsc.load_gather(ugrp, [urow, col])
            acc = acc + vcol * ucol
        sig = 1.0 / (1.0 + jnp.exp(-_SHARP * acc))
        outv[pl.ds(pl.multiple_of(h * _PW, _PW), _PW)] = sig

    fire(0, 0)
    fire(1, 1)

    def step(i, carry):
        h0 = 2 * i
        drain(0)
        compute(h0, 0)

        @pl.when(h0 + 2 < _NPASS)
        def _():
            fire(h0 + 2, 0)

        drain(1)
        compute(h0 + 1, 1)

        @pl.when(h0 + 3 < _NPASS)
        def _():
            fire(h0 + 3, 1)

        return carry

    lax.fori_loop(0, _NPASS // 2, step, 0)

    pltpu.sync_copy(outv, out.at[pl.ds(base, _BPW)])


def kernel(uEmb, vEmb, rec, u):
    rec = rec.astype(jnp.int32)
    u = u.astype(jnp.int32)
    mesh = plsc.VectorSubcoreMesh(core_axis_name="c", subcore_axis_name="s")
    f = pl.kernel(
        _body,
        mesh=mesh,
        out_type=jax.ShapeDtypeStruct((_B,), jnp.float32),
        scratch_types=[
            pltpu.VMEM((_BPW,), jnp.int32),
            pltpu.VMEM((_BPW,), jnp.int32),
            pltpu.VMEM((2 * _SETROWS, _D), jnp.float32),
            pltpu.VMEM((2 * _SETROWS, _D), jnp.float32),
            pltpu.VMEM((_BPW,), jnp.float32),
            pltpu.SemaphoreType.DMA,
            pltpu.SemaphoreType.DMA,
            pltpu.SemaphoreType.DMA,
            pltpu.SemaphoreType.DMA,
        ],
        compiler_params=pltpu.CompilerParams(
            needs_layout_passes=False, use_tc_tiling_on_sc=True),
    )
    return f(uEmb, vEmb, rec, u)
